# baseline (speedup 1.0000x reference)
import functools

import jax
import jax.numpy as jnp
from jax import lax
from jax.experimental import pallas as pl
from jax.experimental.pallas import tpu as pltpu
from jax.experimental.pallas import tpu_sc as plsc

B = 1024
N_FEAT_ROWS = 10000
K = 32
D = 512
JC = 33
JG = 40
NBUF = 4
NC = 2
NS = 16
NW = NC * NS
BPW = B // NW
NP = BPW // 2
L = 16


def _weights_body(nb_ref, nd_ref, w_ref):
    nb = nb_ref[...]
    nd = nd_ref[...]
    self_match = (nb == nd).astype(jnp.float32)
    cnt = self_match
    for k in range(K):
        cnt = cnt + (nb == nb[:, k:k + 1]).astype(jnp.float32)
    inv_nb = 1.0 / cnt
    cnt_self = 1.0 + jnp.sum(self_match, axis=1, keepdims=True)
    inv_self = 1.0 / cnt_self
    c = jnp.sum(inv_nb, axis=1, keepdims=True) + inv_self
    w_ref[...] = jnp.concatenate([inv_nb / c, inv_self / c], axis=1)


_weights = pl.pallas_call(
    _weights_body,
    out_shape=jax.ShapeDtypeStruct((B, JC), jnp.float32),
)


@functools.partial(
    pl.kernel,
    out_type=jax.ShapeDtypeStruct((B, D), jnp.float32),
    mesh=plsc.VectorSubcoreMesh(core_axis_name="c", subcore_axis_name="s"),
    compiler_params=pltpu.CompilerParams(needs_layout_passes=False),
    scratch_types=[
        pltpu.VMEM((BPW * JG,), jnp.int32),
        pltpu.VMEM((BPW * JC * L,), jnp.float32),
        pltpu.VMEM((NBUF, JG, D // 2), jnp.int32),
        pltpu.VMEM((BPW, D), jnp.float32),
        pltpu.SemaphoreType.DMA,
        pltpu.SemaphoreType.DMA,
        pltpu.SemaphoreType.DMA,
        pltpu.SemaphoreType.DMA,
    ],
)
def _sc_aggregate(feat_hbm, ids_hbm, w_hbm, out_hbm,
                  ids_v, w_v, rows_v, obuf_v, sem0, sem1, sem2, sem3):
    sems = [sem0, sem1, sem2, sem3]
    wid = lax.axis_index("s") * NC + lax.axis_index("c")
    base = wid * BPW
    pltpu.sync_copy(ids_hbm.at[pl.ds(base * JG, BPW * JG)], ids_v)
    pltpu.sync_copy(w_hbm.at[pl.ds(base * JC * L, BPW * JC * L)], w_v)

    def gather(r, buf):
        pltpu.async_copy(
            feat_hbm.at[ids_v.at[pl.ds(r * JG, JG)]],
            rows_v.at[buf], sems[buf])

    def gather_wait(buf):
        pltpu.make_async_copy(
            feat_hbm.at[ids_v.at[pl.ds(0, JG)]], rows_v.at[buf], sems[buf]
        ).wait()

    himask = jnp.full((L,), -65536, jnp.int32)

    def compute(r, buf):
        wvs = [w_v[pl.ds((r * JC + j) * L, L)] for j in range(JC)]

        def cc_body(cc, c3):
            off = cc * L
            acc_e = None
            acc_o = None
            for j in range(JC):
                x = rows_v[buf, j, pl.ds(off, L)]
                e = plsc.bitcast(x << 16, jnp.float32)
                o = plsc.bitcast(x & himask, jnp.float32)
                if acc_e is None:
                    acc_e = wvs[j] * e
                    acc_o = wvs[j] * o
                else:
                    acc_e = acc_e + wvs[j] * e
                    acc_o = acc_o + wvs[j] * o
            obuf_v[r, pl.ds(2 * off, L)] = acc_e
            obuf_v[r, pl.ds(2 * off + L, L)] = acc_o
            return c3

        lax.fori_loop(0, D // (2 * L), cc_body, 0)

    for b in range(NBUF - 1):
        gather(b, b)

    def group_body(q, carry):
        for b in range(NBUF):
            r = NBUF * q + b

            @pl.when(r < BPW - (NBUF - 1))
            def _():
                gather(r + NBUF - 1, (b + NBUF - 1) % NBUF)

            gather_wait(b)
            compute(r, b)
        return carry

    lax.fori_loop(0, BPW // NBUF, group_body, 0)
    pltpu.sync_copy(obuf_v, out_hbm.at[pl.ds(base, BPW)])


def kernel(raw_features, nodes, neighbors):
    nb = neighbors.astype(jnp.int32)
    nd = nodes.astype(jnp.int32).reshape(B, 1)
    npad = JG - JC
    pads = (jnp.arange(B, dtype=jnp.int32)[:, None] * npad
            + jnp.arange(npad, dtype=jnp.int32)[None, :]) % N_FEAT_ROWS
    ids = jnp.concatenate([nb, nd, pads], axis=1).reshape(B * JG)
    w = _weights(nb, nd)
    w_exp = jnp.broadcast_to(w[:, :, None], (B, JC, L)).reshape(B * JC * L)
    feat_bf = raw_features.astype(jnp.bfloat16)
    feat_pk = jax.lax.bitcast_convert_type(
        feat_bf.reshape(N_FEAT_ROWS, D // 2, 2), jnp.int32)
    out = _sc_aggregate(feat_pk, ids, w_exp)
    return out.reshape(B, D // 32, 2, L).transpose(0, 1, 3, 2).reshape(B, D)


# speedup vs baseline: 1.8474x; 1.8474x over previous
import functools

import jax
import jax.numpy as jnp
from jax import lax
from jax.experimental import pallas as pl
from jax.experimental.pallas import tpu as pltpu
from jax.experimental.pallas import tpu_sc as plsc

B = 1024
N_FEAT_ROWS = 10000
K = 32
D = 512
JC = 33
JG = 40
NC = 2
NS = 16
NW = NC * NS
L = 16
NBUF = 4

RTC = 512
BSC = B - RTC
BPW = BSC // NW

NBLK = 1000
NSTEPS = N_FEAT_ROWS // NBLK


def _weights_body(nb_ref, nd_ref, w_ref):
    nb = nb_ref[...]
    nd = nd_ref[...]
    self_match = (nb == nd).astype(jnp.float32)
    cnt = self_match
    for k in range(K):
        cnt = cnt + (nb == nb[:, k:k + 1]).astype(jnp.float32)
    inv_nb = 1.0 / cnt
    cnt_self = 1.0 + jnp.sum(self_match, axis=1, keepdims=True)
    inv_self = 1.0 / cnt_self
    c = jnp.sum(inv_nb, axis=1, keepdims=True) + inv_self
    w_ref[...] = jnp.concatenate([inv_nb / c, inv_self / c], axis=1)


_weights = pl.pallas_call(
    _weights_body,
    out_shape=jax.ShapeDtypeStruct((B, JC), jnp.float32),
)


def _dense_body(ids_ref, w_ref, feat_ref, out_ref):
    k = pl.program_id(0)
    ids = ids_ref[...]
    w = w_ref[...]
    n0 = k * NBLK
    iota = lax.broadcasted_iota(jnp.int32, (RTC, NBLK), 1) + n0
    m = jnp.zeros((RTC, NBLK), jnp.float32)
    for j in range(JC):
        m = m + (ids[:, j:j + 1] == iota).astype(jnp.float32) * w[:, j:j + 1]
    part = jnp.dot(m, feat_ref[...], preferred_element_type=jnp.float32)

    @pl.when(k == 0)
    def _():
        out_ref[...] = part

    @pl.when(k > 0)
    def _():
        out_ref[...] = out_ref[...] + part


_dense = pl.pallas_call(
    _dense_body,
    grid=(NSTEPS,),
    in_specs=[
        pl.BlockSpec((RTC, JC), lambda k: (0, 0)),
        pl.BlockSpec((RTC, JC), lambda k: (0, 0)),
        pl.BlockSpec((NBLK, D), lambda k: (k, 0)),
    ],
    out_specs=pl.BlockSpec((RTC, D), lambda k: (0, 0)),
    out_shape=jax.ShapeDtypeStruct((RTC, D), jnp.float32),
)


@functools.partial(
    pl.kernel,
    out_type=jax.ShapeDtypeStruct((BSC, D), jnp.float32),
    mesh=plsc.VectorSubcoreMesh(core_axis_name="c", subcore_axis_name="s"),
    scratch_types=[
        pltpu.VMEM((BPW * JG,), jnp.int32),
        pltpu.VMEM((BPW * JC * L,), jnp.float32),
        pltpu.VMEM((NBUF, JG, D), jnp.float32),
        pltpu.VMEM((BPW, D), jnp.float32),
        pltpu.SemaphoreType.DMA,
        pltpu.SemaphoreType.DMA,
        pltpu.SemaphoreType.DMA,
        pltpu.SemaphoreType.DMA,
    ],
)
def _sc_aggregate(feat_hbm, ids_hbm, w_hbm, out_hbm,
                  ids_v, w_v, rows_v, obuf_v, sem0, sem1, sem2, sem3):
    sems = [sem0, sem1, sem2, sem3]
    wid = lax.axis_index("s") * NC + lax.axis_index("c")
    base = wid * BPW
    pltpu.sync_copy(ids_hbm.at[pl.ds(base * JG, BPW * JG)], ids_v)
    pltpu.sync_copy(w_hbm.at[pl.ds(base * JC * L, BPW * JC * L)], w_v)

    def gather(r, buf):
        pltpu.async_copy(
            feat_hbm.at[ids_v.at[pl.ds(r * JG, JG)]],
            rows_v.at[buf], sems[buf])

    def gather_wait(buf):
        pltpu.make_async_copy(
            feat_hbm.at[ids_v.at[pl.ds(0, JG)]], rows_v.at[buf], sems[buf]
        ).wait()

    def compute(r, buf):
        wvs = [w_v[pl.ds((r * JC + j) * L, L)] for j in range(JC)]

        def cc_body(cc, c3):
            off = cc * L
            acc = wvs[0] * rows_v[buf, 0, pl.ds(off, L)]
            for j in range(1, JC):
                acc = acc + wvs[j] * rows_v[buf, j, pl.ds(off, L)]
            obuf_v[r, pl.ds(off, L)] = acc
            return c3

        lax.fori_loop(0, D // L, cc_body, 0)

    for b in range(NBUF - 1):
        gather(b, b)

    def group_body(q, carry):
        for b in range(NBUF):
            r = NBUF * q + b

            @pl.when(r < BPW - (NBUF - 1))
            def _():
                gather(r + NBUF - 1, (b + NBUF - 1) % NBUF)

            gather_wait(b)
            compute(r, b)
        return carry

    lax.fori_loop(0, BPW // NBUF, group_body, 0)
    pltpu.sync_copy(obuf_v, out_hbm.at[pl.ds(base, BPW)])


def kernel(raw_features, nodes, neighbors):
    nb = neighbors.astype(jnp.int32)
    nd = nodes.astype(jnp.int32).reshape(B, 1)
    w = _weights(nb, nd)
    ids33 = jnp.concatenate([nb, nd], axis=1)

    npad = JG - JC
    pads = (jnp.arange(BSC, dtype=jnp.int32)[:, None] * npad
            + jnp.arange(npad, dtype=jnp.int32)[None, :]) % N_FEAT_ROWS
    ids_sc = jnp.concatenate(
        [ids33[RTC:], pads], axis=1).reshape(BSC * JG)
    w_sc = jnp.broadcast_to(
        w[RTC:, :, None], (BSC, JC, L)).reshape(BSC * JC * L)

    sc_out = _sc_aggregate(raw_features, ids_sc, w_sc)
    tc_out = _dense(ids33[:RTC], w[:RTC], raw_features)
    return jnp.concatenate([tc_out, sc_out], axis=0)


# speedup vs baseline: 3.3708x; 1.8246x over previous
import functools

import jax
import jax.numpy as jnp
from jax import lax
from jax.experimental import pallas as pl
from jax.experimental.pallas import tpu as pltpu
from jax.experimental.pallas import tpu_sc as plsc

B = 1024
N_FEAT_ROWS = 10000
K = 32
D = 512
JC = 33
JG = 40
NC = 2
NS = 16
NW = NC * NS
L = 16
NBUF = 4

BSC = B
BPW = BSC // NW


def _weights_body(nb_ref, nd_ref, w_ref):
    nb = nb_ref[...]
    nd = nd_ref[...]
    self_match = (nb == nd).astype(jnp.float32)
    cnt = self_match
    for k in range(K):
        cnt = cnt + (nb == nb[:, k:k + 1]).astype(jnp.float32)
    inv_nb = 1.0 / cnt
    cnt_self = 1.0 + jnp.sum(self_match, axis=1, keepdims=True)
    inv_self = 1.0 / cnt_self
    c = jnp.sum(inv_nb, axis=1, keepdims=True) + inv_self
    w_ref[...] = jnp.concatenate([inv_nb / c, inv_self / c], axis=1)


_weights = pl.pallas_call(
    _weights_body,
    out_shape=jax.ShapeDtypeStruct((B, JC), jnp.float32),
)


@functools.partial(
    pl.kernel,
    out_type=jax.ShapeDtypeStruct((BSC, D), jnp.float32),
    mesh=plsc.VectorSubcoreMesh(core_axis_name="c", subcore_axis_name="s"),
    scratch_types=[
        pltpu.VMEM((BPW * JG,), jnp.int32),
        pltpu.VMEM((BPW * JC * L,), jnp.float32),
        pltpu.VMEM((NBUF, JG, D), jnp.float32),
        pltpu.VMEM((BPW, D), jnp.float32),
        pltpu.SemaphoreType.DMA,
        pltpu.SemaphoreType.DMA,
        pltpu.SemaphoreType.DMA,
        pltpu.SemaphoreType.DMA,
    ],
)
def _sc_aggregate(feat_hbm, ids_hbm, w_hbm, out_hbm,
                  ids_v, w_v, rows_v, obuf_v, sem0, sem1, sem2, sem3):
    sems = [sem0, sem1, sem2, sem3]
    wid = lax.axis_index("s") * NC + lax.axis_index("c")
    base = wid * BPW
    pltpu.sync_copy(ids_hbm.at[pl.ds(base * JG, BPW * JG)], ids_v)
    pltpu.sync_copy(w_hbm.at[pl.ds(base * JC * L, BPW * JC * L)], w_v)

    def gather(r, buf):
        pltpu.async_copy(
            feat_hbm.at[ids_v.at[pl.ds(r * JG, JG)]],
            rows_v.at[buf], sems[buf])

    def gather_wait(buf):
        pltpu.make_async_copy(
            feat_hbm.at[ids_v.at[pl.ds(0, JG)]], rows_v.at[buf], sems[buf]
        ).wait()

    def compute(r, buf):
        wvs = [w_v[pl.ds((r * JC + j) * L, L)] for j in range(JC)]

        def cc_body(cc, c3):
            off = cc * L
            acc0 = wvs[0] * rows_v[buf, 0, pl.ds(off, L)]
            acc1 = wvs[1] * rows_v[buf, 1, pl.ds(off, L)]
            for j in range(2, JC, 2):
                acc0 = acc0 + wvs[j] * rows_v[buf, j, pl.ds(off, L)]
            for j in range(3, JC, 2):
                acc1 = acc1 + wvs[j] * rows_v[buf, j, pl.ds(off, L)]
            obuf_v[r, pl.ds(off, L)] = acc0 + acc1
            return c3

        lax.fori_loop(0, D // L, cc_body, 0)

    for b in range(NBUF - 1):
        gather(b, b)

    def group_body(q, carry):
        for b in range(NBUF):
            r = NBUF * q + b

            @pl.when(r < BPW - (NBUF - 1))
            def _():
                gather(r + NBUF - 1, (b + NBUF - 1) % NBUF)

            gather_wait(b)
            compute(r, b)
        return carry

    lax.fori_loop(0, BPW // NBUF, group_body, 0)
    pltpu.sync_copy(obuf_v, out_hbm.at[pl.ds(base, BPW)])


def kernel(raw_features, nodes, neighbors):
    nb = neighbors.astype(jnp.int32)
    nd = nodes.astype(jnp.int32).reshape(B, 1)
    w = _weights(nb, nd)
    ids33 = jnp.concatenate([nb, nd], axis=1)

    npad = JG - JC
    pads = (jnp.arange(BSC, dtype=jnp.int32)[:, None] * npad
            + jnp.arange(npad, dtype=jnp.int32)[None, :]) % N_FEAT_ROWS
    ids_sc = jnp.concatenate([ids33, pads], axis=1).reshape(BSC * JG)
    w_sc = jnp.broadcast_to(
        w[:, :, None], (BSC, JC, L)).reshape(BSC * JC * L)

    return _sc_aggregate(raw_features, ids_sc, w_sc)
